# baseline (speedup 1.0000x reference)
import math

import jax
import jax.numpy as jnp
from jax.experimental import pallas as pl
from jax.experimental.pallas import tpu as pltpu

_GELU_C = math.sqrt(2.0 / math.pi)
_LAYER_KS = [10, 3, 3, 3, 3, 2, 2]
_LAYER_SS = [5, 2, 2, 2, 2, 2, 2]


def _gelu(x):
    return 0.5 * x * (1.0 + jnp.tanh(_GELU_C * (x + 0.044715 * x * x * x)))


def _store_act(dst_ref, base, y, t_out, tsz, n_chunks):
    for c in range(n_chunks):
        o = base + c * tsz
        dst_ref[o:o + t_out, :] = y[:, c * 128:(c + 1) * 128]


def _forward_one(g, xm_ref, w0_ref, bias0_ref, g0_ref, b0_ref,
                 layer_refs, o_ref, s0_ref, s1_ref, t_outs, tszs, eps):
    n_chunks = o_ref.shape[1] // 128
    base = (g * n_chunks * tszs[0], g * n_chunks * tszs[1])
    t0 = t_outs[0]
    xr = xm_ref[g]
    xcat = jnp.concatenate([xr[0:t0, :], xr[1:t0 + 1, :]], axis=1)
    acc = jnp.dot(xcat, w0_ref[...], preferred_element_type=jnp.float32)
    acc = acc + bias0_ref[...]
    ones_row = jnp.full((1, t0), 1.0, jnp.float32)
    cat = jnp.concatenate([acc, acc * acc], axis=1)
    sums = jnp.dot(ones_row, cat, preferred_element_type=jnp.float32)
    mean = sums[:, :acc.shape[1]] * (1.0 / t0)
    var = sums[:, acc.shape[1]:] * (1.0 / t0) - mean * mean
    scale = jax.lax.rsqrt(var + eps) * g0_ref[...]
    shift = b0_ref[...] - mean * scale
    y = acc * scale + shift
    y = _gelu(y.astype(jnp.bfloat16)).astype(jnp.float32)
    _store_act(s0_ref, base[0], y, t0, tszs[0], n_chunks)

    bufs = [(s0_ref, tszs[0], base[0]), (s1_ref, tszs[1], base[1])]
    cur = 0
    for i, (w_ref, b_ref) in enumerate(layer_refs):
        k, s = _LAYER_KS[i + 1], _LAYER_SS[i + 1]
        t_out = t_outs[i + 1]
        src, src_tsz, src_base = bufs[cur]
        parts = [src[pl.ds(src_base + c * src_tsz + j, t_out, s), :]
                 for j in range(k) for c in range(n_chunks)]
        xk = jnp.concatenate(parts, axis=1).astype(jnp.bfloat16)
        acc = jnp.dot(xk, w_ref[...], preferred_element_type=jnp.float32)
        if i == len(layer_refs) - 1:
            o_ref[g, :, :] = jnp.transpose(_gelu(acc + b_ref[...]), (1, 0))
        else:
            y = _gelu(acc + b_ref[...])
            dst, dst_tsz, dst_base = bufs[1 - cur]
            _store_act(dst, dst_base, y, t_out, dst_tsz, n_chunks)
            cur = 1 - cur


def _fe_kernel(xm_ref, w0_ref, bias0_ref, g0_ref, b0_ref,
               w1_ref, bias1_ref, w2_ref, bias2_ref, w3_ref, bias3_ref,
               w4_ref, bias4_ref, w5_ref, bias5_ref, w6_ref, bias6_ref,
               o_ref, s0_ref, s1_ref, *, t_outs, tszs, eps, n_batch):
    layer_refs = [(w1_ref, bias1_ref), (w2_ref, bias2_ref), (w3_ref, bias3_ref),
                  (w4_ref, bias4_ref), (w5_ref, bias5_ref), (w6_ref, bias6_ref)]
    for g in range(n_batch):
        _forward_one(g, xm_ref, w0_ref, bias0_ref, g0_ref, b0_ref,
                     layer_refs, o_ref, s0_ref, s1_ref, t_outs, tszs, eps)


def kernel(x, g0, b0, w0, bias0, w1, bias1, w2, bias2, w3, bias3,
           w4, bias4, w5, bias5, w6, bias6):
    B, T = x.shape
    C = w0.shape[0]
    t_outs = []
    t = T
    for k, s in zip(_LAYER_KS, _LAYER_SS):
        t = (t - k) // s + 1
        t_outs.append(t)
    t0, t_last = t_outs[0], t_outs[-1]
    k0, s0 = _LAYER_KS[0], _LAYER_SS[0]

    assert T % s0 == 0 and k0 == 2 * s0
    xm = x.reshape(B, T // s0, s0)

    w0m = jnp.transpose(w0[:, 0, :], (1, 0))

    def prep_w(w):
        k = w.shape[2]
        return jnp.transpose(w, (2, 1, 0)).reshape(k * C, C).astype(jnp.bfloat16)

    def row(v):
        return v.reshape(1, C).astype(jnp.float32)

    ws = [prep_w(w) for w in (w1, w2, w3, w4, w5, w6)]
    bs = [row(v) for v in (bias1, bias2, bias3, bias4, bias5, bias6)]

    def const_map(b):
        return (0, 0)

    def r8(n):
        return (n + 7) // 8 * 8

    gb = 1
    in_specs = [pl.BlockSpec((gb, T // s0, s0), lambda b: (b, 0, 0)),
                pl.BlockSpec((k0, C), const_map),
                pl.BlockSpec((1, C), const_map),
                pl.BlockSpec((1, C), const_map),
                pl.BlockSpec((1, C), const_map)]
    operands = [xm, w0m, row(bias0), row(g0), row(b0)]
    for w, bv in zip(ws, bs):
        in_specs.append(pl.BlockSpec(w.shape, const_map))
        in_specs.append(pl.BlockSpec((1, C), const_map))
        operands.append(w)
        operands.append(bv)

    tszs = (r8(t0), r8(t_outs[1]))
    n_chunks = C // 128
    out = pl.pallas_call(
        lambda *refs: _fe_kernel(*refs, t_outs=t_outs, tszs=tszs, eps=1e-5,
                                 n_batch=gb),
        out_shape=jax.ShapeDtypeStruct((B, C, t_last), jnp.float32),
        grid=(B // gb,),
        in_specs=in_specs,
        out_specs=pl.BlockSpec((gb, C, t_last), lambda b: (b, 0, 0)),
        scratch_shapes=[pltpu.VMEM((gb * n_chunks * tszs[0], 128), jnp.float32),
                        pltpu.VMEM((gb * n_chunks * tszs[1], 128), jnp.float32)],
        compiler_params=pltpu.CompilerParams(
            dimension_semantics=("parallel",),
            vmem_limit_bytes=48 * 1024 * 1024),
    )(*operands)

    return out


# speedup vs baseline: 1.0251x; 1.0251x over previous
import math

import jax
import jax.numpy as jnp
from jax.experimental import pallas as pl
from jax.experimental.pallas import tpu as pltpu

_GELU_C = math.sqrt(2.0 / math.pi)
_LAYER_KS = [10, 3, 3, 3, 3, 2, 2]
_LAYER_SS = [5, 2, 2, 2, 2, 2, 2]


def _gelu(x):
    return 0.5 * x * (1.0 + jnp.tanh(_GELU_C * (x + 0.044715 * x * x * x)))


def _store_act(dst_ref, base, y, t_out, tsz, n_chunks):
    for c in range(n_chunks):
        o = base + c * tsz
        dst_ref[o:o + t_out, :] = y[:, c * 128:(c + 1) * 128]


def _forward_one(g, xm_ref, w0_ref, bias0_ref, g0_ref, b0_ref,
                 layer_refs, o_ref, s0_ref, s1_ref, t_outs, tszs, eps):
    n_chunks = o_ref.shape[1] // 128
    base = (g * n_chunks * tszs[0], g * n_chunks * tszs[1])
    t0 = t_outs[0]
    xr = xm_ref[g]
    xcat = jnp.concatenate([xr[0:t0, :], xr[1:t0 + 1, :]], axis=1)
    acc = jnp.dot(xcat, w0_ref[...], preferred_element_type=jnp.float32)
    acc = acc + bias0_ref[...]
    s1 = jnp.sum(acc, axis=0, keepdims=True)
    s2 = jnp.sum(acc * acc, axis=0, keepdims=True)
    mean = s1 * (1.0 / t0)
    var = s2 * (1.0 / t0) - mean * mean
    scale = jax.lax.rsqrt(var + eps) * g0_ref[...]
    shift = b0_ref[...] - mean * scale
    y = acc * scale + shift
    y = _gelu(y.astype(jnp.bfloat16)).astype(jnp.float32)
    _store_act(s0_ref, base[0], y, t0, tszs[0], n_chunks)

    bufs = [(s0_ref, tszs[0], base[0]), (s1_ref, tszs[1], base[1])]
    cur = 0
    for i, (w_ref, b_ref) in enumerate(layer_refs):
        k, s = _LAYER_KS[i + 1], _LAYER_SS[i + 1]
        t_out = t_outs[i + 1]
        src, src_tsz, src_base = bufs[cur]
        parts = [src[pl.ds(src_base + c * src_tsz + j, t_out, s), :]
                 for j in range(k) for c in range(n_chunks)]
        xk = jnp.concatenate(parts, axis=1).astype(jnp.bfloat16)
        acc = jnp.dot(xk, w_ref[...], preferred_element_type=jnp.float32)
        if i == len(layer_refs) - 1:
            o_ref[g, :, :] = jnp.transpose(_gelu(acc + b_ref[...]), (1, 0))
        else:
            y = _gelu(acc + b_ref[...])
            dst, dst_tsz, dst_base = bufs[1 - cur]
            _store_act(dst, dst_base, y, t_out, dst_tsz, n_chunks)
            cur = 1 - cur


def _fe_kernel(xm_ref, w0_ref, bias0_ref, g0_ref, b0_ref,
               w1_ref, bias1_ref, w2_ref, bias2_ref, w3_ref, bias3_ref,
               w4_ref, bias4_ref, w5_ref, bias5_ref, w6_ref, bias6_ref,
               o_ref, s0_ref, s1_ref, *, t_outs, tszs, eps, n_batch):
    layer_refs = [(w1_ref, bias1_ref), (w2_ref, bias2_ref), (w3_ref, bias3_ref),
                  (w4_ref, bias4_ref), (w5_ref, bias5_ref), (w6_ref, bias6_ref)]
    for g in range(n_batch):
        _forward_one(g, xm_ref, w0_ref, bias0_ref, g0_ref, b0_ref,
                     layer_refs, o_ref, s0_ref, s1_ref, t_outs, tszs, eps)


def kernel(x, g0, b0, w0, bias0, w1, bias1, w2, bias2, w3, bias3,
           w4, bias4, w5, bias5, w6, bias6):
    B, T = x.shape
    C = w0.shape[0]
    t_outs = []
    t = T
    for k, s in zip(_LAYER_KS, _LAYER_SS):
        t = (t - k) // s + 1
        t_outs.append(t)
    t0, t_last = t_outs[0], t_outs[-1]
    k0, s0 = _LAYER_KS[0], _LAYER_SS[0]

    assert T % s0 == 0 and k0 == 2 * s0
    xm = x.reshape(B, T // s0, s0)

    w0m = jnp.transpose(w0[:, 0, :], (1, 0))

    def prep_w(w):
        k = w.shape[2]
        return jnp.transpose(w, (2, 1, 0)).reshape(k * C, C).astype(jnp.bfloat16)

    def row(v):
        return v.reshape(1, C).astype(jnp.float32)

    ws = [prep_w(w) for w in (w1, w2, w3, w4, w5, w6)]
    bs = [row(v) for v in (bias1, bias2, bias3, bias4, bias5, bias6)]

    def const_map(b):
        return (0, 0)

    def r8(n):
        return (n + 7) // 8 * 8

    gb = 1
    in_specs = [pl.BlockSpec((gb, T // s0, s0), lambda b: (b, 0, 0)),
                pl.BlockSpec((k0, C), const_map),
                pl.BlockSpec((1, C), const_map),
                pl.BlockSpec((1, C), const_map),
                pl.BlockSpec((1, C), const_map)]
    operands = [xm, w0m, row(bias0), row(g0), row(b0)]
    for w, bv in zip(ws, bs):
        in_specs.append(pl.BlockSpec(w.shape, const_map))
        in_specs.append(pl.BlockSpec((1, C), const_map))
        operands.append(w)
        operands.append(bv)

    tszs = (r8(t0), r8(t_outs[1]))
    n_chunks = C // 128
    out = pl.pallas_call(
        lambda *refs: _fe_kernel(*refs, t_outs=t_outs, tszs=tszs, eps=1e-5,
                                 n_batch=gb),
        out_shape=jax.ShapeDtypeStruct((B, C, t_last), jnp.float32),
        grid=(B // gb,),
        in_specs=in_specs,
        out_specs=pl.BlockSpec((gb, C, t_last), lambda b: (b, 0, 0)),
        scratch_shapes=[pltpu.VMEM((gb * n_chunks * tszs[0], 128), jnp.float32),
                        pltpu.VMEM((gb * n_chunks * tszs[1], 128), jnp.float32)],
        compiler_params=pltpu.CompilerParams(
            dimension_semantics=("parallel",),
            vmem_limit_bytes=48 * 1024 * 1024),
    )(*operands)

    return out


# speedup vs baseline: 1.0369x; 1.0115x over previous
import math

import jax
import jax.numpy as jnp
from jax.experimental import pallas as pl
from jax.experimental.pallas import tpu as pltpu

_GELU_C = math.sqrt(2.0 / math.pi)
_LAYER_KS = [10, 3, 3, 3, 3, 2, 2]
_LAYER_SS = [5, 2, 2, 2, 2, 2, 2]


def _gelu(x):
    return 0.5 * x * (1.0 + jnp.tanh(_GELU_C * (x + 0.044715 * x * x * x)))


def _store_act(dst_ref, base, y, t_out, tsz, n_chunks):
    for c in range(n_chunks):
        o = base + c * tsz
        dst_ref[o:o + t_out, :] = y[:, c * 128:(c + 1) * 128]


def _forward_one(g, xm_ref, w0_ref, bias0_ref, g0_ref, b0_ref,
                 layer_refs, o_ref, s0_ref, s1_ref, t_outs, tszs, eps):
    n_chunks = o_ref.shape[2] // 128
    base = (0, 0)
    t0 = t_outs[0]
    xr = xm_ref[g]
    xcat = jnp.concatenate([xr[0:t0, :], xr[1:t0 + 1, :]], axis=1)
    acc = jnp.dot(xcat, w0_ref[...], preferred_element_type=jnp.float32)
    acc = acc + bias0_ref[...]
    s1 = jnp.sum(acc, axis=0, keepdims=True)
    s2 = jnp.sum(acc * acc, axis=0, keepdims=True)
    mean = s1 * (1.0 / t0)
    var = s2 * (1.0 / t0) - mean * mean
    scale = jax.lax.rsqrt(var + eps) * g0_ref[...]
    shift = b0_ref[...] - mean * scale
    y = acc * scale + shift
    y = _gelu(y.astype(jnp.bfloat16)).astype(jnp.float32)
    _store_act(s0_ref, base[0], y, t0, tszs[0], n_chunks)

    bufs = [(s0_ref, tszs[0], base[0]), (s1_ref, tszs[1], base[1])]
    cur = 0
    for i, (w_ref, b_ref) in enumerate(layer_refs):
        k, s = _LAYER_KS[i + 1], _LAYER_SS[i + 1]
        t_out = t_outs[i + 1]
        src, src_tsz, src_base = bufs[cur]
        parts = [src[pl.ds(src_base + c * src_tsz + j, t_out, s), :]
                 for j in range(k) for c in range(n_chunks)]
        xk = jnp.concatenate(parts, axis=1).astype(w_ref.dtype)
        acc = jnp.dot(xk, w_ref[...], preferred_element_type=jnp.float32)
        if i == len(layer_refs) - 1:
            o_ref[g, :, :] = _gelu(acc + b_ref[...])
        else:
            y = _gelu(acc + b_ref[...])
            dst, dst_tsz, dst_base = bufs[1 - cur]
            _store_act(dst, dst_base, y, t_out, dst_tsz, n_chunks)
            cur = 1 - cur


def _fe_kernel(xm_ref, w0_ref, bias0_ref, g0_ref, b0_ref,
               w1_ref, bias1_ref, w2_ref, bias2_ref, w3_ref, bias3_ref,
               w4_ref, bias4_ref, w5_ref, bias5_ref, w6_ref, bias6_ref,
               o_ref, *scratch, t_outs, tszs, eps, n_batch):
    layer_refs = [(w1_ref, bias1_ref), (w2_ref, bias2_ref), (w3_ref, bias3_ref),
                  (w4_ref, bias4_ref), (w5_ref, bias5_ref), (w6_ref, bias6_ref)]
    for g in range(n_batch):
        _forward_one(g, xm_ref, w0_ref, bias0_ref, g0_ref, b0_ref,
                     layer_refs, o_ref, scratch[2 * g], scratch[2 * g + 1],
                     t_outs, tszs, eps)


def kernel(x, g0, b0, w0, bias0, w1, bias1, w2, bias2, w3, bias3,
           w4, bias4, w5, bias5, w6, bias6):
    B, T = x.shape
    C = w0.shape[0]
    t_outs = []
    t = T
    for k, s in zip(_LAYER_KS, _LAYER_SS):
        t = (t - k) // s + 1
        t_outs.append(t)
    t0, t_last = t_outs[0], t_outs[-1]
    k0, s0 = _LAYER_KS[0], _LAYER_SS[0]

    assert T % s0 == 0 and k0 == 2 * s0
    xm = x.reshape(B, T // s0, s0)

    w0m = jnp.transpose(w0[:, 0, :], (1, 0))

    def prep_w(w, dt):
        k = w.shape[2]
        return jnp.transpose(w, (2, 1, 0)).reshape(k * C, C).astype(dt)

    def row(v):
        return v.reshape(1, C).astype(jnp.float32)

    ws = [prep_w(w, jnp.bfloat16) for w in (w1, w2, w3, w4)]
    ws += [prep_w(w, jnp.float32) for w in (w5, w6)]
    bs = [row(v) for v in (bias1, bias2, bias3, bias4, bias5, bias6)]

    def const_map(b):
        return (0, 0)

    def r8(n):
        return (n + 7) // 8 * 8

    gb = 1
    in_specs = [pl.BlockSpec((gb, T // s0, s0), lambda b: (b, 0, 0)),
                pl.BlockSpec((k0, C), const_map),
                pl.BlockSpec((1, C), const_map),
                pl.BlockSpec((1, C), const_map),
                pl.BlockSpec((1, C), const_map)]
    operands = [xm, w0m, row(bias0), row(g0), row(b0)]
    for w, bv in zip(ws, bs):
        in_specs.append(pl.BlockSpec(w.shape, const_map))
        in_specs.append(pl.BlockSpec((1, C), const_map))
        operands.append(w)
        operands.append(bv)

    tszs = (r8(t0), r8(t_outs[1]))
    n_chunks = C // 128
    out = pl.pallas_call(
        lambda *refs: _fe_kernel(*refs, t_outs=t_outs, tszs=tszs, eps=1e-5,
                                 n_batch=gb),
        out_shape=jax.ShapeDtypeStruct((B, t_last, C), jnp.float32),
        grid=(B // gb,),
        in_specs=in_specs,
        out_specs=pl.BlockSpec((gb, t_last, C), lambda b: (b, 0, 0)),
        scratch_shapes=[pltpu.VMEM((n_chunks * tszs[0], 128), jnp.float32),
                        pltpu.VMEM((n_chunks * tszs[1], 128), jnp.float32)] * gb,
        compiler_params=pltpu.CompilerParams(
            dimension_semantics=("parallel",),
            vmem_limit_bytes=48 * 1024 * 1024),
    )(*operands)

    return jnp.transpose(out, (0, 2, 1))
